# baseline (speedup 1.0000x reference)
import jax
import jax.numpy as jnp
from jax import lax
from jax.experimental import pallas as pl
from jax.experimental.pallas import tpu as pltpu, tpu_sc as plsc

N = 100000
E = 3200000
D = 32
H = N // 2
HPAD = 50176
STRIPE = HPAD // 16
ZROWS = 98
C = 400
C1 = 8000
EPT = E // 16
BN = 1000

_SC_PARAMS = pltpu.CompilerParams(needs_layout_passes=False,
                                  use_tc_tiling_on_sc=False)


def _ab_body(nf_ref, wst_ref, wdt_ref, a_ref, b_ref):
    x = nf_ref[...]
    a_ref[...] = jnp.dot(x, wst_ref[...], preferred_element_type=jnp.float32)
    b_ref[...] = jnp.dot(x, wdt_ref[...], preferred_element_type=jnp.float32)


def _compute_ab(nf, WsT, WdT):
    return pl.pallas_call(
        _ab_body,
        grid=(N // BN,),
        in_specs=[
            pl.BlockSpec((BN, D), lambda i: (i, 0)),
            pl.BlockSpec((D, D), lambda i: (0, 0)),
            pl.BlockSpec((D, D), lambda i: (0, 0)),
        ],
        out_specs=[
            pl.BlockSpec((BN, D), lambda i: (i, 0)),
            pl.BlockSpec((BN, D), lambda i: (i, 0)),
        ],
        out_shape=[
            jax.ShapeDtypeStruct((N, D), jnp.float32),
            jax.ShapeDtypeStruct((N, D), jnp.float32),
        ],
    )(nf, WsT, WdT)


def _wsum_body(dst_h, ef_h, parts_o, wred_o, dstv, efv, wsumv, acc, tmp, sem):
    c = lax.axis_index("c")
    s = lax.axis_index("s")
    cH = c * H
    zero16 = jnp.zeros((16,), jnp.float32)

    def _zw(i, carry):
        wsumv[pl.ds(i * 16, 16)] = zero16
        return carry
    lax.fori_loop(0, HPAD // 16, _zw, 0)

    def _chunk(i, carry):
        base = s * EPT + i * C1
        pltpu.sync_copy(dst_h.at[pl.ds(base, C1)], dstv)
        pltpu.sync_copy(ef_h.at[pl.ds(base, C1)], efv)

        @plsc.parallel_loop(0, C1 // 16, 1, unroll=4)
        def _grp(k):
            sl = pl.ds(k * 16, 16)
            wk = 1.0 / (efv[sl] + 1e-5)
            dl = dstv[sl] - cH
            own = jnp.logical_and(dl >= 0, dl < H)
            dlf = jnp.where(own, dl, H)
            plsc.addupdate_scatter(wsumv, [dlf], wk)
        return carry
    lax.fori_loop(0, EPT // C1, _chunk, 0)

    pltpu.sync_copy(wsumv, parts_o.at[pl.ds((c * 16 + s) * HPAD, HPAD)])
    plsc.subcore_barrier()

    off = s * STRIPE
    pltpu.sync_copy(parts_o.at[pl.ds(c * 16 * HPAD + off, STRIPE)], acc)

    def _red(t, carry):
        pltpu.sync_copy(parts_o.at[pl.ds((c * 16 + t) * HPAD + off, STRIPE)],
                        tmp)

        def _add(k, carry2):
            sl = pl.ds(k * 16, 16)
            acc[sl] = acc[sl] + tmp[sl]
            return carry2
        lax.fori_loop(0, STRIPE // 16, _add, 0)
        return carry
    lax.fori_loop(1, 16, _red, 0)
    pltpu.sync_copy(acc, wred_o.at[pl.ds(c * HPAD + off, STRIPE)])


def _run_wsum(dst, ef):
    mesh = plsc.VectorSubcoreMesh(core_axis_name="c", subcore_axis_name="s")
    fn = pl.kernel(
        _wsum_body,
        out_type=[
            jax.ShapeDtypeStruct((2 * 16 * HPAD,), jnp.float32),
            jax.ShapeDtypeStruct((2 * HPAD,), jnp.float32),
        ],
        mesh=mesh,
        compiler_params=_SC_PARAMS,
        scratch_types=[
            pltpu.VMEM((C1,), jnp.int32),
            pltpu.VMEM((C1,), jnp.float32),
            pltpu.VMEM((HPAD,), jnp.float32),
            pltpu.VMEM((STRIPE,), jnp.float32),
            pltpu.VMEM((STRIPE,), jnp.float32),
            pltpu.SemaphoreType.DMA,
        ],
    )
    return fn(dst, ef)


def _agg_body(a_h, b_h, src_h, dst_h, ef_h, we_h, agg_o,
              aggs, zs, pk0, dlocv0, g0, pk1, dlocv1, g1, wev,
              semg0, semg1, semsc0, semsc1):
    c = lax.axis_index("c")
    s = lax.axis_index("s")
    cH = c * H
    zero16 = jnp.zeros((16,), jnp.float32)

    def _zz(i, carry):
        g0[i, pl.ds(0, 16)] = zero16
        g0[i, pl.ds(16, 16)] = zero16
        return carry
    lax.fori_loop(0, C, _zz, 0)
    for j in range(8):
        pltpu.sync_copy(g0.at[pl.ds(0, STRIPE // 8)],
                        aggs.at[pl.ds(s * STRIPE + j * (STRIPE // 8),
                                      STRIPE // 8)])

    @pl.when(s == 0)
    def _():
        pltpu.sync_copy(g0, zs)

    pltpu.sync_copy(we_h, wev)
    we0 = wev[pl.ds(0, 16)]
    we1 = wev[pl.ds(16, 16)]

    plsc.subcore_barrier()

    bufs = ((pk0, dlocv0, g0, semg0, semsc0),
            (pk1, dlocv1, g1, semg1, semsc1))

    def idx_load(i, b):
        pk, dlocv, g, semg, semsc = b
        base = s * EPT + i * C
        pltpu.sync_copy(src_h.at[pl.ds(base, C)], pk.at[pl.ds(0, C)])
        pltpu.sync_copy(dst_h.at[pl.ds(base, C)], pk.at[pl.ds(C, C)])
        pltpu.sync_copy(ef_h.at[pl.ds(base, C)], pk.at[pl.ds(2 * C, C)])

    def fire_a(b):
        pk, dlocv, g, semg, semsc = b
        pltpu.async_copy(a_h.at[pk.at[pl.ds(0, C)]], g, semg, add=True)

    def fire_b(b):
        pk, dlocv, g, semg, semsc = b
        pltpu.async_copy(b_h.at[pk.at[pl.ds(C, C)]], g, semg, add=True)

    def fire_zero(b):
        pk, dlocv, g, semg, semsc = b
        pltpu.async_copy(zs, g, semg)

    def wait_zero(b):
        pk, dlocv, g, semg, semsc = b
        pltpu.make_async_copy(zs, g, semg).wait()

    def wait_g(b):
        pk, dlocv, g, semg, semsc = b
        pltpu.make_async_copy(a_h.at[pk.at[pl.ds(0, C)]], g, semg).wait()

    def fire_scatter(b):
        pk, dlocv, g, semg, semsc = b
        pltpu.async_copy(g, aggs.at[dlocv], semsc, add=True)

    def wait_scatter(b):
        pk, dlocv, g, semg, semsc = b
        pltpu.make_async_copy(g, aggs.at[dlocv], semsc).wait()

    def compute(b):
        pk, dlocv, g, semg, semsc = b

        @plsc.parallel_loop(0, C // 16, 1)
        def _grp(k):
            sl = pl.ds(2 * C + k * 16, 16)
            efk = plsc.bitcast(pk[sl], jnp.float32)
            wk = 1.0 / (efk + 1e-5)
            dl = pk[pl.ds(C + k * 16, 16)] - cH
            own = jnp.logical_and(dl >= 0, dl < H)
            dlf = jnp.where(own, dl, H)
            dlocv[pl.ds(k * 16, 16)] = dlf
            for i in range(16):
                e = k * 16 + i
                ef_s = efk[i]
                w_s = wk[i]
                ga = g[e, pl.ds(0, 16)]
                gb = g[e, pl.ds(16, 16)]
                g[e, pl.ds(0, 16)] = jnp.maximum(ga + we0 * ef_s, 0.0) * w_s
                g[e, pl.ds(16, 16)] = jnp.maximum(gb + we1 * ef_s, 0.0) * w_s

    pltpu.sync_copy(zs, g1)
    idx_load(0, bufs[0])
    fire_a(bufs[0])
    fire_b(bufs[0])
    idx_load(1, bufs[1])
    fire_a(bufs[1])
    fire_b(bufs[1])

    def _it(j, carry):
        a_i = 2 * j
        wait_g(bufs[0])
        wait_g(bufs[0])
        compute(bufs[0])
        fire_scatter(bufs[0])
        wait_g(bufs[1])
        wait_g(bufs[1])
        compute(bufs[1])
        fire_scatter(bufs[1])

        @pl.when(j < (EPT // C) // 2 - 1)
        def _():
            wait_scatter(bufs[0])
            fire_zero(bufs[0])
            idx_load(a_i + 2, bufs[0])
            wait_zero(bufs[0])
            fire_a(bufs[0])
            fire_b(bufs[0])
            wait_scatter(bufs[1])
            fire_zero(bufs[1])
            idx_load(a_i + 3, bufs[1])
            wait_zero(bufs[1])
            fire_a(bufs[1])
            fire_b(bufs[1])
        return carry
    lax.fori_loop(0, (EPT // C) // 2, _it, 0)

    wait_scatter(bufs[0])
    wait_scatter(bufs[1])
    plsc.subcore_barrier()

    pltpu.sync_copy(aggs.at[pl.ds(s * STRIPE, STRIPE)],
                    agg_o.at[c, pl.ds(s * STRIPE, STRIPE)])


def _run_agg(A, B, src, dst, ef, we):
    mesh = plsc.VectorSubcoreMesh(core_axis_name="c", subcore_axis_name="s")
    fn = pl.kernel(
        _agg_body,
        out_type=[
            jax.ShapeDtypeStruct((2, HPAD, D), jnp.float32),
        ],
        mesh=mesh,
        compiler_params=_SC_PARAMS,
        scratch_types=[
            pltpu.VMEM_SHARED((HPAD, D), jnp.float32),
            pltpu.VMEM_SHARED((C, D), jnp.float32),
            pltpu.VMEM((3 * C,), jnp.int32),
            pltpu.VMEM((C,), jnp.int32),
            pltpu.VMEM((C, D), jnp.float32),
            pltpu.VMEM((3 * C,), jnp.int32),
            pltpu.VMEM((C,), jnp.int32),
            pltpu.VMEM((C, D), jnp.float32),
            pltpu.VMEM((D,), jnp.float32),
            pltpu.SemaphoreType.DMA,
            pltpu.SemaphoreType.DMA,
            pltpu.SemaphoreType.DMA,
            pltpu.SemaphoreType.DMA,
        ],
    )
    return fn(A, B, src, dst, lax.bitcast_convert_type(ef, jnp.int32), we)


def _node_body(nf_ref, nt_ref, agg_ref, ws_ref, w0_ref, w1_ref, w2_ref,
               out_ref):
    aggraw = agg_ref[0]
    wsum = ws_ref[0]
    good = wsum > 0.0
    aggn = jnp.where(good, aggraw / jnp.where(good, wsum, 1.0), 0.0)
    h = jnp.concatenate([nf_ref[...], aggn], axis=1)
    y0 = jnp.maximum(jnp.dot(h, w0_ref[...], preferred_element_type=jnp.float32), 0.0)
    y1 = jnp.maximum(jnp.dot(h, w1_ref[...], preferred_element_type=jnp.float32), 0.0)
    y2 = jnp.maximum(jnp.dot(h, w2_ref[...], preferred_element_type=jnp.float32), 0.0)
    nt = nt_ref[...]
    out_ref[...] = jnp.where(nt == 0, y0, jnp.where(nt == 1, y1, y2))


def _node_update(nf, node_type2, agg2, wred3, W0T, W1T, W2T):
    nb = H // BN
    return pl.pallas_call(
        _node_body,
        grid=(N // BN,),
        in_specs=[
            pl.BlockSpec((BN, D), lambda i: (i, 0)),
            pl.BlockSpec((BN, 1), lambda i: (i, 0)),
            pl.BlockSpec((1, BN, D), lambda i: (i // nb, i % nb, 0)),
            pl.BlockSpec((1, BN, 1), lambda i: (i // nb, i % nb, 0)),
            pl.BlockSpec((2 * D, D), lambda i: (0, 0)),
            pl.BlockSpec((2 * D, D), lambda i: (0, 0)),
            pl.BlockSpec((2 * D, D), lambda i: (0, 0)),
        ],
        out_specs=pl.BlockSpec((BN, D), lambda i: (i, 0)),
        out_shape=jax.ShapeDtypeStruct((N, D), jnp.float32),
    )(nf, node_type2, agg2, wred3, W0T, W1T, W2T)


def kernel(nf, edge_index, init_ef, node_type, W_edge, W_node0, W_node1,
           W_node2):
    src = edge_index[0]
    dst = edge_index[1]
    ef = init_ef[:, 0]
    WsT = W_edge[:, :D].T
    WdT = W_edge[:, D:2 * D].T
    we = W_edge[:, 2 * D]

    A, B = _compute_ab(nf, WsT, WdT)
    _parts, wred = _run_wsum(dst, ef)
    agg2, = _run_agg(A, B, src, dst, ef, we)

    node_type2 = node_type.reshape(N, 1)
    wred3 = wred.reshape(2, HPAD, 1)
    return _node_update(nf, node_type2, agg2, wred3,
                        W_node0.T, W_node1.T, W_node2.T)


# speedup vs baseline: 1.1027x; 1.1027x over previous
import jax
import jax.numpy as jnp
from jax import lax
from jax.experimental import pallas as pl
from jax.experimental.pallas import tpu as pltpu, tpu_sc as plsc

N = 100000
E = 3200000
D = 32
H = N // 2
HPAD = 50176
STRIPE = HPAD // 16
ZROWS = 98
C = 400
C1 = 8000
EPT = E // 16
BN = 1000

_SC_PARAMS = pltpu.CompilerParams(needs_layout_passes=False,
                                  use_tc_tiling_on_sc=False)


def _ab_body(nf_ref, wst_ref, wdt_ref, a_ref, b_ref):
    x = nf_ref[...]
    a_ref[...] = jnp.dot(x, wst_ref[...], preferred_element_type=jnp.float32)
    b_ref[...] = jnp.dot(x, wdt_ref[...], preferred_element_type=jnp.float32)


def _compute_ab(nf, WsT, WdT):
    return pl.pallas_call(
        _ab_body,
        grid=(N // BN,),
        in_specs=[
            pl.BlockSpec((BN, D), lambda i: (i, 0)),
            pl.BlockSpec((D, D), lambda i: (0, 0)),
            pl.BlockSpec((D, D), lambda i: (0, 0)),
        ],
        out_specs=[
            pl.BlockSpec((BN, D), lambda i: (i, 0)),
            pl.BlockSpec((BN, D), lambda i: (i, 0)),
        ],
        out_shape=[
            jax.ShapeDtypeStruct((N, D), jnp.float32),
            jax.ShapeDtypeStruct((N, D), jnp.float32),
        ],
    )(nf, WsT, WdT)


def _wsum_body(dst_h, ef_h, parts_o, wred_o, dstv, efv, wsumv, acc, tmp, sem):
    c = lax.axis_index("c")
    s = lax.axis_index("s")
    cH = c * H
    zero16 = jnp.zeros((16,), jnp.float32)

    def _zw(i, carry):
        wsumv[pl.ds(i * 16, 16)] = zero16
        return carry
    lax.fori_loop(0, HPAD // 16, _zw, 0)

    def _chunk(i, carry):
        base = s * EPT + i * C1
        pltpu.sync_copy(dst_h.at[pl.ds(base, C1)], dstv)
        pltpu.sync_copy(ef_h.at[pl.ds(base, C1)], efv)

        @plsc.parallel_loop(0, C1 // 16, 1, unroll=4)
        def _grp(k):
            sl = pl.ds(k * 16, 16)
            wk = 1.0 / (efv[sl] + 1e-5)
            dl = dstv[sl] - cH
            own = jnp.logical_and(dl >= 0, dl < H)
            dlf = jnp.where(own, dl, H)
            plsc.addupdate_scatter(wsumv, [dlf], wk)
        return carry
    lax.fori_loop(0, EPT // C1, _chunk, 0)

    pltpu.sync_copy(wsumv, parts_o.at[pl.ds((c * 16 + s) * HPAD, HPAD)])
    plsc.subcore_barrier()

    off = s * STRIPE
    pltpu.sync_copy(parts_o.at[pl.ds(c * 16 * HPAD + off, STRIPE)], acc)

    def _red(t, carry):
        pltpu.sync_copy(parts_o.at[pl.ds((c * 16 + t) * HPAD + off, STRIPE)],
                        tmp)

        def _add(k, carry2):
            sl = pl.ds(k * 16, 16)
            acc[sl] = acc[sl] + tmp[sl]
            return carry2
        lax.fori_loop(0, STRIPE // 16, _add, 0)
        return carry
    lax.fori_loop(1, 16, _red, 0)
    pltpu.sync_copy(acc, wred_o.at[pl.ds(c * HPAD + off, STRIPE)])


def _run_wsum(dst, ef):
    mesh = plsc.VectorSubcoreMesh(core_axis_name="c", subcore_axis_name="s")
    fn = pl.kernel(
        _wsum_body,
        out_type=[
            jax.ShapeDtypeStruct((2 * 16 * HPAD,), jnp.float32),
            jax.ShapeDtypeStruct((2 * HPAD,), jnp.float32),
        ],
        mesh=mesh,
        compiler_params=_SC_PARAMS,
        scratch_types=[
            pltpu.VMEM((C1,), jnp.int32),
            pltpu.VMEM((C1,), jnp.float32),
            pltpu.VMEM((HPAD,), jnp.float32),
            pltpu.VMEM((STRIPE,), jnp.float32),
            pltpu.VMEM((STRIPE,), jnp.float32),
            pltpu.SemaphoreType.DMA,
        ],
    )
    return fn(dst, ef)


def _agg_body(a_h, b_h, src_h, dst_h, ef_h, we_h, agg_o,
              aggs, pk0, dlocv0, g0, pk1, dlocv1, g1, wev,
              semg0, semg1, semsc0, semsc1):
    c = lax.axis_index("c")
    s = lax.axis_index("s")
    cH = c * H
    zero16 = jnp.zeros((16,), jnp.float32)

    def _zz(i, carry):
        g0[i, pl.ds(0, 16)] = zero16
        g0[i, pl.ds(16, 16)] = zero16
        return carry
    lax.fori_loop(0, C, _zz, 0)
    for j in range(8):
        pltpu.sync_copy(g0.at[pl.ds(0, STRIPE // 8)],
                        aggs.at[pl.ds(s * STRIPE + j * (STRIPE // 8),
                                      STRIPE // 8)])

    pltpu.sync_copy(we_h, wev)
    we0 = wev[pl.ds(0, 16)]
    we1 = wev[pl.ds(16, 16)]

    plsc.subcore_barrier()

    bufs = ((pk0, dlocv0, g0, semg0, semsc0),
            (pk1, dlocv1, g1, semg1, semsc1))

    def idx_load(i, b):
        pk, dlocv, g, semg, semsc = b
        base = s * EPT + i * C
        pltpu.sync_copy(src_h.at[pl.ds(base, C)], pk.at[pl.ds(0, C)])
        pltpu.sync_copy(dst_h.at[pl.ds(base, C)], pk.at[pl.ds(C, C)])
        pltpu.sync_copy(ef_h.at[pl.ds(base, C)], pk.at[pl.ds(2 * C, C)])

    def fire_a(b):
        pk, dlocv, g, semg, semsc = b
        pltpu.async_copy(a_h.at[pk.at[pl.ds(0, C)]], g, semg)

    def fire_b(b):
        pk, dlocv, g, semg, semsc = b
        pltpu.async_copy(b_h.at[pk.at[pl.ds(C, C)]], g, semg, add=True)

    def wait_g(b):
        pk, dlocv, g, semg, semsc = b
        pltpu.make_async_copy(a_h.at[pk.at[pl.ds(0, C)]], g, semg).wait()

    def fire_scatter(b):
        pk, dlocv, g, semg, semsc = b
        pltpu.async_copy(g, aggs.at[dlocv], semsc, add=True)

    def wait_scatter(b):
        pk, dlocv, g, semg, semsc = b
        pltpu.make_async_copy(g, aggs.at[dlocv], semsc).wait()

    def compute(b):
        pk, dlocv, g, semg, semsc = b

        @plsc.parallel_loop(0, C // 16, 1)
        def _grp(k):
            sl = pl.ds(2 * C + k * 16, 16)
            efk = plsc.bitcast(pk[sl], jnp.float32)
            wk = 1.0 / (efk + 1e-5)
            dl = pk[pl.ds(C + k * 16, 16)] - cH
            own = jnp.logical_and(dl >= 0, dl < H)
            dlf = jnp.where(own, dl, H)
            dlocv[pl.ds(k * 16, 16)] = dlf
            for i in range(16):
                e = k * 16 + i
                ef_s = efk[i]
                w_s = wk[i]
                ga = g[e, pl.ds(0, 16)]
                gb = g[e, pl.ds(16, 16)]
                g[e, pl.ds(0, 16)] = jnp.maximum(ga + we0 * ef_s, 0.0) * w_s
                g[e, pl.ds(16, 16)] = jnp.maximum(gb + we1 * ef_s, 0.0) * w_s

    idx_load(0, bufs[0])
    fire_a(bufs[0])

    def _it(j, carry):
        a_i = 2 * j
        wait_g(bufs[0])
        fire_b(bufs[0])

        @pl.when(j > 0)
        def _():
            wait_scatter(bufs[1])
        idx_load(a_i + 1, bufs[1])
        fire_a(bufs[1])
        wait_g(bufs[0])
        compute(bufs[0])
        fire_scatter(bufs[0])
        wait_g(bufs[1])
        fire_b(bufs[1])

        @pl.when(j < (EPT // C) // 2 - 1)
        def _():
            wait_scatter(bufs[0])
            idx_load(a_i + 2, bufs[0])
            fire_a(bufs[0])
        wait_g(bufs[1])
        compute(bufs[1])
        fire_scatter(bufs[1])
        return carry
    lax.fori_loop(0, (EPT // C) // 2, _it, 0)

    wait_scatter(bufs[0])
    wait_scatter(bufs[1])
    plsc.subcore_barrier()

    pltpu.sync_copy(aggs.at[pl.ds(s * STRIPE, STRIPE)],
                    agg_o.at[c, pl.ds(s * STRIPE, STRIPE)])


def _run_agg(A, B, src, dst, ef, we):
    mesh = plsc.VectorSubcoreMesh(core_axis_name="c", subcore_axis_name="s")
    fn = pl.kernel(
        _agg_body,
        out_type=[
            jax.ShapeDtypeStruct((2, HPAD, D), jnp.float32),
        ],
        mesh=mesh,
        compiler_params=_SC_PARAMS,
        scratch_types=[
            pltpu.VMEM_SHARED((HPAD, D), jnp.float32),
            pltpu.VMEM((3 * C,), jnp.int32),
            pltpu.VMEM((C,), jnp.int32),
            pltpu.VMEM((C, D), jnp.float32),
            pltpu.VMEM((3 * C,), jnp.int32),
            pltpu.VMEM((C,), jnp.int32),
            pltpu.VMEM((C, D), jnp.float32),
            pltpu.VMEM((D,), jnp.float32),
            pltpu.SemaphoreType.DMA,
            pltpu.SemaphoreType.DMA,
            pltpu.SemaphoreType.DMA,
            pltpu.SemaphoreType.DMA,
        ],
    )
    return fn(A, B, src, dst, lax.bitcast_convert_type(ef, jnp.int32), we)


def _node_body(nf_ref, nt_ref, agg_ref, ws_ref, w0_ref, w1_ref, w2_ref,
               out_ref):
    aggraw = agg_ref[0]
    wsum = ws_ref[0]
    good = wsum > 0.0
    aggn = jnp.where(good, aggraw / jnp.where(good, wsum, 1.0), 0.0)
    h = jnp.concatenate([nf_ref[...], aggn], axis=1)
    y0 = jnp.maximum(jnp.dot(h, w0_ref[...], preferred_element_type=jnp.float32), 0.0)
    y1 = jnp.maximum(jnp.dot(h, w1_ref[...], preferred_element_type=jnp.float32), 0.0)
    y2 = jnp.maximum(jnp.dot(h, w2_ref[...], preferred_element_type=jnp.float32), 0.0)
    nt = nt_ref[...]
    out_ref[...] = jnp.where(nt == 0, y0, jnp.where(nt == 1, y1, y2))


def _node_update(nf, node_type2, agg2, wred3, W0T, W1T, W2T):
    nb = H // BN
    return pl.pallas_call(
        _node_body,
        grid=(N // BN,),
        in_specs=[
            pl.BlockSpec((BN, D), lambda i: (i, 0)),
            pl.BlockSpec((BN, 1), lambda i: (i, 0)),
            pl.BlockSpec((1, BN, D), lambda i: (i // nb, i % nb, 0)),
            pl.BlockSpec((1, BN, 1), lambda i: (i // nb, i % nb, 0)),
            pl.BlockSpec((2 * D, D), lambda i: (0, 0)),
            pl.BlockSpec((2 * D, D), lambda i: (0, 0)),
            pl.BlockSpec((2 * D, D), lambda i: (0, 0)),
        ],
        out_specs=pl.BlockSpec((BN, D), lambda i: (i, 0)),
        out_shape=jax.ShapeDtypeStruct((N, D), jnp.float32),
    )(nf, node_type2, agg2, wred3, W0T, W1T, W2T)


def kernel(nf, edge_index, init_ef, node_type, W_edge, W_node0, W_node1,
           W_node2):
    src = edge_index[0]
    dst = edge_index[1]
    ef = init_ef[:, 0]
    WsT = W_edge[:, :D].T
    WdT = W_edge[:, D:2 * D].T
    we = W_edge[:, 2 * D]

    A, B = _compute_ab(nf, WsT, WdT)
    _parts, wred = _run_wsum(dst, ef)
    agg2, = _run_agg(A, B, src, dst, ef, we)

    node_type2 = node_type.reshape(N, 1)
    wred3 = wred.reshape(2, HPAD, 1)
    return _node_update(nf, node_type2, agg2, wred3,
                        W_node0.T, W_node1.T, W_node2.T)


# speedup vs baseline: 1.1288x; 1.0236x over previous
import jax
import jax.numpy as jnp
from jax import lax
from jax.experimental import pallas as pl
from jax.experimental.pallas import tpu as pltpu, tpu_sc as plsc

N = 100000
E = 3200000
D = 32
H = N // 2
HPAD = 50176
STRIPE = HPAD // 16
ZROWS = 98
C = 400
C1 = 8000
EPT = E // 16
BN = 2000

_SC_PARAMS = pltpu.CompilerParams(needs_layout_passes=False,
                                  use_tc_tiling_on_sc=False)


def _ab_body(nf_ref, wst_ref, wdt_ref, a_ref, b_ref):
    x = nf_ref[...]
    a_ref[...] = jnp.dot(x, wst_ref[...], preferred_element_type=jnp.float32)
    b_ref[...] = jnp.dot(x, wdt_ref[...], preferred_element_type=jnp.float32)


def _compute_ab(nf, WsT, WdT):
    return pl.pallas_call(
        _ab_body,
        grid=(N // BN,),
        in_specs=[
            pl.BlockSpec((BN, D), lambda i: (i, 0)),
            pl.BlockSpec((D, D), lambda i: (0, 0)),
            pl.BlockSpec((D, D), lambda i: (0, 0)),
        ],
        out_specs=[
            pl.BlockSpec((BN, D), lambda i: (i, 0)),
            pl.BlockSpec((BN, D), lambda i: (i, 0)),
        ],
        out_shape=[
            jax.ShapeDtypeStruct((N, D), jnp.float32),
            jax.ShapeDtypeStruct((N, D), jnp.float32),
        ],
    )(nf, WsT, WdT)


def _wsum_body(dst_h, ef_h, parts_o, wred_o, dstv, efv, wsumv, acc, tmp, sem):
    c = lax.axis_index("c")
    s = lax.axis_index("s")
    cH = c * H
    zero16 = jnp.zeros((16,), jnp.float32)

    def _zw(i, carry):
        wsumv[pl.ds(i * 16, 16)] = zero16
        return carry
    lax.fori_loop(0, HPAD // 16, _zw, 0)

    def _chunk(i, carry):
        base = s * EPT + i * C1
        pltpu.sync_copy(dst_h.at[pl.ds(base, C1)], dstv)
        pltpu.sync_copy(ef_h.at[pl.ds(base, C1)], efv)

        @plsc.parallel_loop(0, C1 // 16, 1, unroll=4)
        def _grp(k):
            sl = pl.ds(k * 16, 16)
            wk = 1.0 / (efv[sl] + 1e-5)
            dl = dstv[sl] - cH
            own = jnp.logical_and(dl >= 0, dl < H)
            dlf = jnp.where(own, dl, H)
            plsc.addupdate_scatter(wsumv, [dlf], wk)
        return carry
    lax.fori_loop(0, EPT // C1, _chunk, 0)

    pltpu.sync_copy(wsumv, parts_o.at[pl.ds((c * 16 + s) * HPAD, HPAD)])
    plsc.subcore_barrier()

    off = s * STRIPE
    pltpu.sync_copy(parts_o.at[pl.ds(c * 16 * HPAD + off, STRIPE)], acc)

    def _red(t, carry):
        pltpu.sync_copy(parts_o.at[pl.ds((c * 16 + t) * HPAD + off, STRIPE)],
                        tmp)

        def _add(k, carry2):
            sl = pl.ds(k * 16, 16)
            acc[sl] = acc[sl] + tmp[sl]
            return carry2
        lax.fori_loop(0, STRIPE // 16, _add, 0)
        return carry
    lax.fori_loop(1, 16, _red, 0)
    pltpu.sync_copy(acc, wred_o.at[pl.ds(c * HPAD + off, STRIPE)])


def _run_wsum(dst, ef):
    mesh = plsc.VectorSubcoreMesh(core_axis_name="c", subcore_axis_name="s")
    fn = pl.kernel(
        _wsum_body,
        out_type=[
            jax.ShapeDtypeStruct((2 * 16 * HPAD,), jnp.float32),
            jax.ShapeDtypeStruct((2 * HPAD,), jnp.float32),
        ],
        mesh=mesh,
        compiler_params=_SC_PARAMS,
        scratch_types=[
            pltpu.VMEM((C1,), jnp.int32),
            pltpu.VMEM((C1,), jnp.float32),
            pltpu.VMEM((HPAD,), jnp.float32),
            pltpu.VMEM((STRIPE,), jnp.float32),
            pltpu.VMEM((STRIPE,), jnp.float32),
            pltpu.SemaphoreType.DMA,
        ],
    )
    return fn(dst, ef)


def _agg_body(a_h, b_h, src_h, dst_h, ef_h, we_h, agg_o,
              aggs, pk0, dlocv0, g0, pk1, dlocv1, g1, wev,
              semg0, semg1, semsc0, semsc1):
    c = lax.axis_index("c")
    s = lax.axis_index("s")
    cH = c * H
    zero16 = jnp.zeros((16,), jnp.float32)

    def _zz(i, carry):
        g0[i, pl.ds(0, 16)] = zero16
        g0[i, pl.ds(16, 16)] = zero16
        return carry
    lax.fori_loop(0, C, _zz, 0)
    for j in range(8):
        pltpu.sync_copy(g0.at[pl.ds(0, STRIPE // 8)],
                        aggs.at[pl.ds(s * STRIPE + j * (STRIPE // 8),
                                      STRIPE // 8)])

    pltpu.sync_copy(we_h, wev)
    we0 = wev[pl.ds(0, 16)]
    we1 = wev[pl.ds(16, 16)]

    plsc.subcore_barrier()

    bufs = ((pk0, dlocv0, g0, semg0, semsc0),
            (pk1, dlocv1, g1, semg1, semsc1))

    def idx_load(i, b):
        pk, dlocv, g, semg, semsc = b
        base = s * EPT + i * C
        pltpu.sync_copy(src_h.at[pl.ds(base, C)], pk.at[pl.ds(0, C)])
        pltpu.sync_copy(dst_h.at[pl.ds(base, C)], pk.at[pl.ds(C, C)])
        pltpu.sync_copy(ef_h.at[pl.ds(base, C)], pk.at[pl.ds(2 * C, C)])

    def fire_a(b):
        pk, dlocv, g, semg, semsc = b
        pltpu.async_copy(a_h.at[pk.at[pl.ds(0, C)]], g, semg)

    def fire_b(b):
        pk, dlocv, g, semg, semsc = b
        pltpu.async_copy(b_h.at[pk.at[pl.ds(C, C)]], g, semg, add=True)

    def wait_g(b):
        pk, dlocv, g, semg, semsc = b
        pltpu.make_async_copy(a_h.at[pk.at[pl.ds(0, C)]], g, semg).wait()

    def fire_scatter(b):
        pk, dlocv, g, semg, semsc = b
        pltpu.async_copy(g, aggs.at[dlocv], semsc, add=True)

    def wait_scatter(b):
        pk, dlocv, g, semg, semsc = b
        pltpu.make_async_copy(g, aggs.at[dlocv], semsc).wait()

    def compute(b):
        pk, dlocv, g, semg, semsc = b

        @plsc.parallel_loop(0, C // 16, 1)
        def _grp(k):
            sl = pl.ds(2 * C + k * 16, 16)
            efk = plsc.bitcast(pk[sl], jnp.float32)
            wk = 1.0 / (efk + 1e-5)
            dl = pk[pl.ds(C + k * 16, 16)] - cH
            own = jnp.logical_and(dl >= 0, dl < H)
            dlf = jnp.where(own, dl, H)
            dlocv[pl.ds(k * 16, 16)] = dlf
            for i in range(16):
                e = k * 16 + i
                ef_s = efk[i]
                w_s = wk[i]
                ga = g[e, pl.ds(0, 16)]
                gb = g[e, pl.ds(16, 16)]
                g[e, pl.ds(0, 16)] = jnp.maximum(ga + we0 * ef_s, 0.0) * w_s
                g[e, pl.ds(16, 16)] = jnp.maximum(gb + we1 * ef_s, 0.0) * w_s

    idx_load(0, bufs[0])
    fire_a(bufs[0])

    def _it(j, carry):
        a_i = 2 * j
        wait_g(bufs[0])
        fire_b(bufs[0])

        @pl.when(j > 0)
        def _():
            wait_scatter(bufs[1])
        idx_load(a_i + 1, bufs[1])
        fire_a(bufs[1])
        wait_g(bufs[0])
        compute(bufs[0])
        fire_scatter(bufs[0])
        wait_g(bufs[1])
        fire_b(bufs[1])

        @pl.when(j < (EPT // C) // 2 - 1)
        def _():
            wait_scatter(bufs[0])
            idx_load(a_i + 2, bufs[0])
            fire_a(bufs[0])
        wait_g(bufs[1])
        compute(bufs[1])
        fire_scatter(bufs[1])
        return carry
    lax.fori_loop(0, (EPT // C) // 2, _it, 0)

    wait_scatter(bufs[0])
    wait_scatter(bufs[1])
    plsc.subcore_barrier()

    pltpu.sync_copy(aggs.at[pl.ds(s * STRIPE, STRIPE)],
                    agg_o.at[c, pl.ds(s * STRIPE, STRIPE)])


def _run_agg(A, B, src, dst, ef, we):
    mesh = plsc.VectorSubcoreMesh(core_axis_name="c", subcore_axis_name="s")
    fn = pl.kernel(
        _agg_body,
        out_type=[
            jax.ShapeDtypeStruct((2, HPAD, D), jnp.float32),
        ],
        mesh=mesh,
        compiler_params=_SC_PARAMS,
        scratch_types=[
            pltpu.VMEM_SHARED((HPAD, D), jnp.float32),
            pltpu.VMEM((3 * C,), jnp.int32),
            pltpu.VMEM((C,), jnp.int32),
            pltpu.VMEM((C, D), jnp.float32),
            pltpu.VMEM((3 * C,), jnp.int32),
            pltpu.VMEM((C,), jnp.int32),
            pltpu.VMEM((C, D), jnp.float32),
            pltpu.VMEM((D,), jnp.float32),
            pltpu.SemaphoreType.DMA,
            pltpu.SemaphoreType.DMA,
            pltpu.SemaphoreType.DMA,
            pltpu.SemaphoreType.DMA,
        ],
    )
    return fn(A, B, src, dst, lax.bitcast_convert_type(ef, jnp.int32), we)


def _node_body(nf_ref, nt_ref, agg_ref, ws_ref, w0_ref, w1_ref, w2_ref,
               out_ref):
    aggraw = agg_ref[0]
    wsum = ws_ref[0]
    good = wsum > 0.0
    aggn = jnp.where(good, aggraw / jnp.where(good, wsum, 1.0), 0.0)
    h = jnp.concatenate([nf_ref[...], aggn], axis=1)
    y0 = jnp.maximum(jnp.dot(h, w0_ref[...], preferred_element_type=jnp.float32), 0.0)
    y1 = jnp.maximum(jnp.dot(h, w1_ref[...], preferred_element_type=jnp.float32), 0.0)
    y2 = jnp.maximum(jnp.dot(h, w2_ref[...], preferred_element_type=jnp.float32), 0.0)
    nt = nt_ref[...]
    out_ref[...] = jnp.where(nt == 0, y0, jnp.where(nt == 1, y1, y2))


def _node_update(nf, node_type2, agg2, wred3, W0T, W1T, W2T):
    nb = H // BN
    return pl.pallas_call(
        _node_body,
        grid=(N // BN,),
        in_specs=[
            pl.BlockSpec((BN, D), lambda i: (i, 0)),
            pl.BlockSpec((BN, 1), lambda i: (i, 0)),
            pl.BlockSpec((1, BN, D), lambda i: (i // nb, i % nb, 0)),
            pl.BlockSpec((1, BN, 1), lambda i: (i // nb, i % nb, 0)),
            pl.BlockSpec((2 * D, D), lambda i: (0, 0)),
            pl.BlockSpec((2 * D, D), lambda i: (0, 0)),
            pl.BlockSpec((2 * D, D), lambda i: (0, 0)),
        ],
        out_specs=pl.BlockSpec((BN, D), lambda i: (i, 0)),
        out_shape=jax.ShapeDtypeStruct((N, D), jnp.float32),
    )(nf, node_type2, agg2, wred3, W0T, W1T, W2T)


def kernel(nf, edge_index, init_ef, node_type, W_edge, W_node0, W_node1,
           W_node2):
    src = edge_index[0]
    dst = edge_index[1]
    ef = init_ef[:, 0]
    WsT = W_edge[:, :D].T
    WdT = W_edge[:, D:2 * D].T
    we = W_edge[:, 2 * D]

    A, B = _compute_ab(nf, WsT, WdT)
    _parts, wred = _run_wsum(dst, ef)
    agg2, = _run_agg(A, B, src, dst, ef, we)

    node_type2 = node_type.reshape(N, 1)
    wred3 = wred.reshape(2, HPAD, 1)
    return _node_update(nf, node_type2, agg2, wred3,
                        W_node0.T, W_node1.T, W_node2.T)


# speedup vs baseline: 1.1432x; 1.0128x over previous
import jax
import jax.numpy as jnp
from jax import lax
from jax.experimental import pallas as pl
from jax.experimental.pallas import tpu as pltpu, tpu_sc as plsc

N = 100000
E = 3200000
D = 32
H = N // 2
HPAD = 50176
STRIPE = HPAD // 16
ZROWS = 98
C = 400
C1 = 8000
EPT = E // 16
BN = 5000

_SC_PARAMS = pltpu.CompilerParams(needs_layout_passes=False,
                                  use_tc_tiling_on_sc=False)


def _ab_body(nf_ref, wst_ref, wdt_ref, a_ref, b_ref):
    x = nf_ref[...]
    a_ref[...] = jnp.dot(x, wst_ref[...], preferred_element_type=jnp.float32)
    b_ref[...] = jnp.dot(x, wdt_ref[...], preferred_element_type=jnp.float32)


def _compute_ab(nf, WsT, WdT):
    return pl.pallas_call(
        _ab_body,
        grid=(N // BN,),
        in_specs=[
            pl.BlockSpec((BN, D), lambda i: (i, 0)),
            pl.BlockSpec((D, D), lambda i: (0, 0)),
            pl.BlockSpec((D, D), lambda i: (0, 0)),
        ],
        out_specs=[
            pl.BlockSpec((BN, D), lambda i: (i, 0)),
            pl.BlockSpec((BN, D), lambda i: (i, 0)),
        ],
        out_shape=[
            jax.ShapeDtypeStruct((N, D), jnp.float32),
            jax.ShapeDtypeStruct((N, D), jnp.float32),
        ],
    )(nf, WsT, WdT)


def _wsum_body(dst_h, ef_h, parts_o, wred_o, dstv, efv, wsumv, acc, tmp, sem):
    c = lax.axis_index("c")
    s = lax.axis_index("s")
    cH = c * H
    zero16 = jnp.zeros((16,), jnp.float32)

    def _zw(i, carry):
        wsumv[pl.ds(i * 16, 16)] = zero16
        return carry
    lax.fori_loop(0, HPAD // 16, _zw, 0)

    def _chunk(i, carry):
        base = s * EPT + i * C1
        pltpu.sync_copy(dst_h.at[pl.ds(base, C1)], dstv)
        pltpu.sync_copy(ef_h.at[pl.ds(base, C1)], efv)

        @plsc.parallel_loop(0, C1 // 16, 1, unroll=4)
        def _grp(k):
            sl = pl.ds(k * 16, 16)
            wk = 1.0 / (efv[sl] + 1e-5)
            dl = dstv[sl] - cH
            own = jnp.logical_and(dl >= 0, dl < H)
            dlf = jnp.where(own, dl, H)
            plsc.addupdate_scatter(wsumv, [dlf], wk)
        return carry
    lax.fori_loop(0, EPT // C1, _chunk, 0)

    pltpu.sync_copy(wsumv, parts_o.at[pl.ds((c * 16 + s) * HPAD, HPAD)])
    plsc.subcore_barrier()

    off = s * STRIPE
    pltpu.sync_copy(parts_o.at[pl.ds(c * 16 * HPAD + off, STRIPE)], acc)

    def _red(t, carry):
        pltpu.sync_copy(parts_o.at[pl.ds((c * 16 + t) * HPAD + off, STRIPE)],
                        tmp)

        def _add(k, carry2):
            sl = pl.ds(k * 16, 16)
            acc[sl] = acc[sl] + tmp[sl]
            return carry2
        lax.fori_loop(0, STRIPE // 16, _add, 0)
        return carry
    lax.fori_loop(1, 16, _red, 0)
    pltpu.sync_copy(acc, wred_o.at[pl.ds(c * HPAD + off, STRIPE)])


def _run_wsum(dst, ef):
    mesh = plsc.VectorSubcoreMesh(core_axis_name="c", subcore_axis_name="s")
    fn = pl.kernel(
        _wsum_body,
        out_type=[
            jax.ShapeDtypeStruct((2 * 16 * HPAD,), jnp.float32),
            jax.ShapeDtypeStruct((2 * HPAD,), jnp.float32),
        ],
        mesh=mesh,
        compiler_params=_SC_PARAMS,
        scratch_types=[
            pltpu.VMEM((C1,), jnp.int32),
            pltpu.VMEM((C1,), jnp.float32),
            pltpu.VMEM((HPAD,), jnp.float32),
            pltpu.VMEM((STRIPE,), jnp.float32),
            pltpu.VMEM((STRIPE,), jnp.float32),
            pltpu.SemaphoreType.DMA,
        ],
    )
    return fn(dst, ef)


def _agg_body(a_h, b_h, src_h, dst_h, ef_h, we_h, agg_o,
              aggs, pk0, dlocv0, g0, pk1, dlocv1, g1, wev,
              semg0, semg1, semsc0, semsc1):
    c = lax.axis_index("c")
    s = lax.axis_index("s")
    cH = c * H
    zero16 = jnp.zeros((16,), jnp.float32)

    def _zz(i, carry):
        g0[i, pl.ds(0, 16)] = zero16
        g0[i, pl.ds(16, 16)] = zero16
        return carry
    lax.fori_loop(0, C, _zz, 0)
    for j in range(8):
        pltpu.sync_copy(g0.at[pl.ds(0, STRIPE // 8)],
                        aggs.at[pl.ds(s * STRIPE + j * (STRIPE // 8),
                                      STRIPE // 8)])

    pltpu.sync_copy(we_h, wev)
    we0 = wev[pl.ds(0, 16)]
    we1 = wev[pl.ds(16, 16)]

    plsc.subcore_barrier()

    bufs = ((pk0, dlocv0, g0, semg0, semsc0),
            (pk1, dlocv1, g1, semg1, semsc1))

    def idx_load(i, b):
        pk, dlocv, g, semg, semsc = b
        base = s * EPT + i * C
        pltpu.sync_copy(src_h.at[pl.ds(base, C)], pk.at[pl.ds(0, C)])
        pltpu.sync_copy(dst_h.at[pl.ds(base, C)], pk.at[pl.ds(C, C)])
        pltpu.sync_copy(ef_h.at[pl.ds(base, C)], pk.at[pl.ds(2 * C, C)])

    def fire_a(b):
        pk, dlocv, g, semg, semsc = b
        pltpu.async_copy(a_h.at[pk.at[pl.ds(0, C)]], g, semg)

    def fire_b(b):
        pk, dlocv, g, semg, semsc = b
        pltpu.async_copy(b_h.at[pk.at[pl.ds(C, C)]], g, semg, add=True)

    def wait_g(b):
        pk, dlocv, g, semg, semsc = b
        pltpu.make_async_copy(a_h.at[pk.at[pl.ds(0, C)]], g, semg).wait()

    def fire_scatter(b):
        pk, dlocv, g, semg, semsc = b
        pltpu.async_copy(g, aggs.at[dlocv], semsc, add=True)

    def wait_scatter(b):
        pk, dlocv, g, semg, semsc = b
        pltpu.make_async_copy(g, aggs.at[dlocv], semsc).wait()

    def compute(b):
        pk, dlocv, g, semg, semsc = b

        @plsc.parallel_loop(0, C // 16, 1)
        def _grp(k):
            sl = pl.ds(2 * C + k * 16, 16)
            efk = plsc.bitcast(pk[sl], jnp.float32)
            wk = 1.0 / (efk + 1e-5)
            dl = pk[pl.ds(C + k * 16, 16)] - cH
            own = jnp.logical_and(dl >= 0, dl < H)
            dlf = jnp.where(own, dl, H)
            dlocv[pl.ds(k * 16, 16)] = dlf
            for i in range(16):
                e = k * 16 + i
                ef_s = efk[i]
                w_s = wk[i]
                ga = g[e, pl.ds(0, 16)]
                gb = g[e, pl.ds(16, 16)]
                g[e, pl.ds(0, 16)] = jnp.maximum(ga + we0 * ef_s, 0.0) * w_s
                g[e, pl.ds(16, 16)] = jnp.maximum(gb + we1 * ef_s, 0.0) * w_s

    idx_load(0, bufs[0])
    fire_a(bufs[0])

    def _it(j, carry):
        a_i = 2 * j
        wait_g(bufs[0])
        fire_b(bufs[0])

        @pl.when(j > 0)
        def _():
            wait_scatter(bufs[1])
        idx_load(a_i + 1, bufs[1])
        fire_a(bufs[1])
        wait_g(bufs[0])
        compute(bufs[0])
        fire_scatter(bufs[0])
        wait_g(bufs[1])
        fire_b(bufs[1])

        @pl.when(j < (EPT // C) // 2 - 1)
        def _():
            wait_scatter(bufs[0])
            idx_load(a_i + 2, bufs[0])
            fire_a(bufs[0])
        wait_g(bufs[1])
        compute(bufs[1])
        fire_scatter(bufs[1])
        return carry
    lax.fori_loop(0, (EPT // C) // 2, _it, 0)

    wait_scatter(bufs[0])
    wait_scatter(bufs[1])
    plsc.subcore_barrier()

    pltpu.sync_copy(aggs.at[pl.ds(s * STRIPE, STRIPE)],
                    agg_o.at[c, pl.ds(s * STRIPE, STRIPE)])


def _run_agg(A, B, src, dst, ef, we):
    mesh = plsc.VectorSubcoreMesh(core_axis_name="c", subcore_axis_name="s")
    fn = pl.kernel(
        _agg_body,
        out_type=[
            jax.ShapeDtypeStruct((2, HPAD, D), jnp.float32),
        ],
        mesh=mesh,
        compiler_params=_SC_PARAMS,
        scratch_types=[
            pltpu.VMEM_SHARED((HPAD, D), jnp.float32),
            pltpu.VMEM((3 * C,), jnp.int32),
            pltpu.VMEM((C,), jnp.int32),
            pltpu.VMEM((C, D), jnp.float32),
            pltpu.VMEM((3 * C,), jnp.int32),
            pltpu.VMEM((C,), jnp.int32),
            pltpu.VMEM((C, D), jnp.float32),
            pltpu.VMEM((D,), jnp.float32),
            pltpu.SemaphoreType.DMA,
            pltpu.SemaphoreType.DMA,
            pltpu.SemaphoreType.DMA,
            pltpu.SemaphoreType.DMA,
        ],
    )
    return fn(A, B, src, dst, lax.bitcast_convert_type(ef, jnp.int32), we)


def _node_body(nf_ref, nt_ref, agg_ref, ws_ref, w0_ref, w1_ref, w2_ref,
               out_ref):
    aggraw = agg_ref[0]
    wsum = ws_ref[0]
    good = wsum > 0.0
    aggn = jnp.where(good, aggraw / jnp.where(good, wsum, 1.0), 0.0)
    h = jnp.concatenate([nf_ref[...], aggn], axis=1)
    y0 = jnp.maximum(jnp.dot(h, w0_ref[...], preferred_element_type=jnp.float32), 0.0)
    y1 = jnp.maximum(jnp.dot(h, w1_ref[...], preferred_element_type=jnp.float32), 0.0)
    y2 = jnp.maximum(jnp.dot(h, w2_ref[...], preferred_element_type=jnp.float32), 0.0)
    nt = nt_ref[...]
    out_ref[...] = jnp.where(nt == 0, y0, jnp.where(nt == 1, y1, y2))


def _node_update(nf, node_type2, agg2, wred3, W0T, W1T, W2T):
    nb = H // BN
    return pl.pallas_call(
        _node_body,
        grid=(N // BN,),
        in_specs=[
            pl.BlockSpec((BN, D), lambda i: (i, 0)),
            pl.BlockSpec((BN, 1), lambda i: (i, 0)),
            pl.BlockSpec((1, BN, D), lambda i: (i // nb, i % nb, 0)),
            pl.BlockSpec((1, BN, 1), lambda i: (i // nb, i % nb, 0)),
            pl.BlockSpec((2 * D, D), lambda i: (0, 0)),
            pl.BlockSpec((2 * D, D), lambda i: (0, 0)),
            pl.BlockSpec((2 * D, D), lambda i: (0, 0)),
        ],
        out_specs=pl.BlockSpec((BN, D), lambda i: (i, 0)),
        out_shape=jax.ShapeDtypeStruct((N, D), jnp.float32),
    )(nf, node_type2, agg2, wred3, W0T, W1T, W2T)


def kernel(nf, edge_index, init_ef, node_type, W_edge, W_node0, W_node1,
           W_node2):
    src = edge_index[0]
    dst = edge_index[1]
    ef = init_ef[:, 0]
    WsT = W_edge[:, :D].T
    WdT = W_edge[:, D:2 * D].T
    we = W_edge[:, 2 * D]

    A, B = _compute_ab(nf, WsT, WdT)
    _parts, wred = _run_wsum(dst, ef)
    agg2, = _run_agg(A, B, src, dst, ef, we)

    node_type2 = node_type.reshape(N, 1)
    wred3 = wred.reshape(2, HPAD, 1)
    return _node_update(nf, node_type2, agg2, wred3,
                        W_node0.T, W_node1.T, W_node2.T)
